# baseline (speedup 1.0000x reference)
import jax
import jax.numpy as jnp
from jax.experimental import pallas as pl
from jax.experimental.pallas import tpu as pltpu


def _chunk_schedule(n, bm):
    chunks = [bm] * (n // bm - 1)
    rem = n - bm * (n // bm - 1)
    while rem > 64:
        c = min(((rem // 2 + 7) // 8) * 8, rem - 8)
        chunks.append(c)
        rem -= c
    chunks.append(rem)
    offs, o = [], 0
    for c in chunks:
        offs.append(o)
        o += c
    return chunks, offs


def _gcn_body(chunks, offs,
              w_ref, x_hbm, adj_hbm, o_hbm,
              x_ref, s_ref, b0, b1, b2, ob0, ob1,
              feat_sem, in_sems, out_sems):
    bufs = [b0, b1, b2]
    obufs = [ob0, ob1]

    feat_cp = pltpu.make_async_copy(x_hbm, x_ref, feat_sem)
    feat_cp.start()

    def in_copy(i):
        rows, st = chunks[i], offs[i]
        return pltpu.make_async_copy(
            adj_hbm.at[pl.ds(st, rows), :],
            bufs[i % 3].at[pl.ds(0, rows), :],
            in_sems.at[i % 3])

    in_cps = [in_copy(i) for i in range(len(chunks))]
    in_cps[0].start()
    in_cps[1].start()

    feat_cp.wait()
    s_ref[...] = jnp.dot(x_ref[...], w_ref[...],
                         preferred_element_type=jnp.float32
                         ).astype(jnp.bfloat16)

    out_cps = [None] * len(chunks)
    for i, rows in enumerate(chunks):
        if i + 2 < len(chunks):
            in_cps[i + 2].start()
        in_cps[i].wait()
        if i >= 2:
            out_cps[i - 2].wait()
        acc = jnp.dot(bufs[i % 3][0:rows, :].astype(jnp.bfloat16),
                      s_ref[...], preferred_element_type=jnp.float32)
        obufs[i % 2][0:rows, :] = jnp.where(acc >= 0, acc, 0.2 * acc)
        cp = pltpu.make_async_copy(
            obufs[i % 2].at[pl.ds(0, rows), :],
            o_hbm.at[pl.ds(offs[i], rows), :],
            out_sems.at[i % 2])
        cp.start()
        out_cps[i] = cp

    out_cps[-2].wait()
    out_cps[-1].wait()


def kernel(features, adj, weight):
    n, din = features.shape
    dout = weight.shape[1]
    bm = 400
    chunks, offs = _chunk_schedule(n, bm)

    def body(*refs):
        _gcn_body(chunks, offs, *refs)

    out = pl.pallas_call(
        body,
        grid=(1,),
        in_specs=[
            pl.BlockSpec((din, dout), lambda i: (0, 0)),
            pl.BlockSpec(memory_space=pl.ANY),
            pl.BlockSpec(memory_space=pl.ANY),
        ],
        out_specs=pl.BlockSpec(memory_space=pl.ANY),
        out_shape=jax.ShapeDtypeStruct((n, dout), jnp.float32),
        scratch_shapes=[
            pltpu.VMEM((n, din), jnp.float32),
            pltpu.VMEM((n, dout), jnp.bfloat16),
            pltpu.VMEM((bm, n), jnp.float32),
            pltpu.VMEM((bm, n), jnp.float32),
            pltpu.VMEM((bm, n), jnp.float32),
            pltpu.VMEM((bm, dout), jnp.float32),
            pltpu.VMEM((bm, dout), jnp.float32),
            pltpu.SemaphoreType.DMA,
            pltpu.SemaphoreType.DMA((3,)),
            pltpu.SemaphoreType.DMA((2,)),
        ],
        compiler_params=pltpu.CompilerParams(
            dimension_semantics=("arbitrary",)),
    )(weight, features, adj)
    return out


# speedup vs baseline: 1.0198x; 1.0198x over previous
import jax
import jax.numpy as jnp
from jax.experimental import pallas as pl
from jax.experimental.pallas import tpu as pltpu


def _taper(rem):
    chunks = []
    while rem > 64:
        c = min(((rem // 2 + 7) // 8) * 8, rem - 8)
        chunks.append(c)
        rem -= c
    chunks.append(rem)
    offs, o = [], 0
    for c in chunks:
        offs.append(o)
        o += c
    return chunks, offs


def _gcn_body(n, din, dout, bm,
              x_ref, w_ref, adjb_ref, adj_hbm, o_hbm,
              s_ref, tailbuf, obuf, tob,
              tail_sems, out_sems, tout_sem):
    i = pl.program_id(0)
    nfull = pl.num_programs(0)
    base = (n // bm - 1) * bm
    chunks, offs = _taper(n - base)
    par = jax.lax.rem(i, 2)

    @pl.when(i == 0)
    def _():
        s_ref[...] = jnp.dot(x_ref[...], w_ref[...],
                             preferred_element_type=jnp.float32
                             ).astype(jnp.bfloat16)

    @pl.when(i == nfull - 2)
    def _():
        for k, (rows, off) in enumerate(zip(chunks, offs)):
            pltpu.make_async_copy(
                adj_hbm.at[pl.ds(base + off, rows), :],
                tailbuf.at[pl.ds(off, rows), :],
                tail_sems.at[k]).start()

    @pl.when(i >= 2)
    def _():
        pltpu.make_async_copy(
            obuf.at[pl.ds(par * bm, bm), :],
            o_hbm.at[pl.ds((i - 2) * bm, bm), :],
            out_sems.at[par]).wait()

    acc = jnp.dot(adjb_ref[...].astype(jnp.bfloat16), s_ref[...],
                  preferred_element_type=jnp.float32)
    obuf[pl.ds(par * bm, bm), :] = jnp.where(acc >= 0, acc, 0.2 * acc)
    pltpu.make_async_copy(
        obuf.at[pl.ds(par * bm, bm), :],
        o_hbm.at[pl.ds(i * bm, bm), :],
        out_sems.at[par]).start()

    @pl.when(i == nfull - 1)
    def _():
        for k, (rows, off) in enumerate(zip(chunks, offs)):
            pltpu.make_async_copy(
                adj_hbm.at[pl.ds(base + off, rows), :],
                tailbuf.at[pl.ds(off, rows), :],
                tail_sems.at[k]).wait()
            part = jnp.dot(tailbuf[off:off + rows, :].astype(jnp.bfloat16),
                           s_ref[...], preferred_element_type=jnp.float32)
            tob[off:off + rows, :] = jnp.where(part >= 0, part, 0.2 * part)
        tcp = pltpu.make_async_copy(
            tob, o_hbm.at[pl.ds(base, n - base), :], tout_sem)
        tcp.start()
        pltpu.make_async_copy(
            obuf.at[pl.ds((1 - par) * bm, bm), :],
            o_hbm.at[pl.ds((nfull - 2) * bm, bm), :],
            out_sems.at[1 - par]).wait()
        pltpu.make_async_copy(
            obuf.at[pl.ds(par * bm, bm), :],
            o_hbm.at[pl.ds((nfull - 1) * bm, bm), :],
            out_sems.at[par]).wait()
        tcp.wait()


def kernel(features, adj, weight):
    n, din = features.shape
    dout = weight.shape[1]
    bm = 400
    nfull = n // bm - 1
    n_tail = len(_taper(n - nfull * bm)[0])

    def body(*refs):
        _gcn_body(n, din, dout, bm, *refs)

    out = pl.pallas_call(
        body,
        grid=(nfull,),
        in_specs=[
            pl.BlockSpec((n, din), lambda i: (0, 0)),
            pl.BlockSpec((din, dout), lambda i: (0, 0)),
            pl.BlockSpec((bm, n), lambda i: (i, 0)),
            pl.BlockSpec(memory_space=pl.ANY),
        ],
        out_specs=pl.BlockSpec(memory_space=pl.ANY),
        out_shape=jax.ShapeDtypeStruct((n, dout), jnp.float32),
        scratch_shapes=[
            pltpu.VMEM((n, dout), jnp.bfloat16),
            pltpu.VMEM((bm, n), jnp.float32),
            pltpu.VMEM((2 * bm, dout), jnp.float32),
            pltpu.VMEM((bm, dout), jnp.float32),
            pltpu.SemaphoreType.DMA((n_tail,)),
            pltpu.SemaphoreType.DMA((2,)),
            pltpu.SemaphoreType.DMA,
        ],
        compiler_params=pltpu.CompilerParams(
            dimension_semantics=("arbitrary",)),
    )(features, weight, adj, adj)
    return out
